# baseline (speedup 1.0000x reference)
import functools

import jax
import jax.numpy as jnp
from jax import lax
from jax.experimental import pallas as pl
from jax.experimental.pallas import tpu as pltpu
from jax.experimental.pallas import tpu_sc as plsc

N = 320000
D = 128
NSEG = 10000
SPAD = 11264
TEMP_INV = 1.0 / 11.313708498984761


_BLK = 125
_NBLK = N // (D * _BLK)


def _tc_scores_body(q_ref, k_ref, o_ref):
    s = jnp.sum(q_ref[...] * k_ref[...], axis=3)
    o_ref[...] = jnp.exp(s * TEMP_INV)


def _tc_scores(q, k):
    q4 = q.reshape(_NBLK, _BLK, D, D)
    k4 = k.reshape(_NBLK, _BLK, D, D)
    out = pl.pallas_call(
        _tc_scores_body,
        grid=(_NBLK,),
        in_specs=[
            pl.BlockSpec((1, _BLK, D, D), lambda i: (i, 0, 0, 0)),
            pl.BlockSpec((1, _BLK, D, D), lambda i: (i, 0, 0, 0)),
        ],
        out_specs=pl.BlockSpec((1, _BLK, D), lambda i: (i, 0, 0)),
        out_shape=jax.ShapeDtypeStruct((_NBLK, _BLK, D), jnp.float32),
    )(q4, k4)
    return out.reshape(N)


_NTILES = 16
_CH = 80
_E_PER_TILE_P1 = N // _NTILES
_E_PER_TILE_P2 = N // (2 * _NTILES)
_P1_CHUNKS = _E_PER_TILE_P1 // _CH
_P2_CHUNKS = _E_PER_TILE_P2 // _CH
_SEG_PER_TILE = SPAD // _NTILES
_GRP = 25
_WCH = 640
_WCAP = 10240


def _sc_softmax_body(e_hbm, idx_hbm, out_hbm,
                     idx1_v, val1_v, win_v, milist_v,
                     idx_v, rng_v, val_v, seg_v, acc_sh,
                     sem_ld, sem_sc, sem_p2):
    c = lax.axis_index("c")
    s = lax.axis_index("s")

    zero16 = jnp.zeros((16,), jnp.float32)

    wid = c * _NTILES + s
    base1 = s * _E_PER_TILE_P1
    base2 = wid * _E_PER_TILE_P2
    pltpu.async_copy(idx_hbm.at[pl.ds(base1, _E_PER_TILE_P1)], idx1_v, sem_ld)
    pltpu.async_copy(e_hbm.at[pl.ds(base1, _E_PER_TILE_P1)], val1_v, sem_ld)
    pltpu.async_copy(idx_hbm.at[pl.ds(base2, _E_PER_TILE_P2)], idx_v, sem_p2)
    pltpu.async_copy(e_hbm.at[pl.ds(base2, _E_PER_TILE_P2)], val_v, sem_p2)

    def _zero_body(i, _):
        seg_v[pl.ds(i * 16, 16)] = zero16
        return _
    lax.fori_loop(0, _SEG_PER_TILE // 16, _zero_body, None)
    pltpu.sync_copy(seg_v, acc_sh.at[pl.ds(s * _SEG_PER_TILE, _SEG_PER_TILE)])
    plsc.subcore_barrier()

    pltpu.make_async_copy(idx_hbm.at[pl.ds(base1, _E_PER_TILE_P1)], idx1_v,
                          sem_ld).wait()
    pltpu.make_async_copy(e_hbm.at[pl.ds(base1, _E_PER_TILE_P1)], val1_v,
                          sem_ld).wait()
    lo1 = jnp.min(idx1_v[pl.ds(0, 16)].astype(jnp.float32)).astype(jnp.int32)
    hi1 = jnp.max(idx1_v[pl.ds(_E_PER_TILE_P1 - 16, 16)]
                  .astype(jnp.float32)).astype(jnp.int32)
    lo18 = jnp.bitwise_and(lo1, -8)
    zero16 = jnp.zeros((16,), jnp.float32)

    nz = ((hi1 - lo18 + 128) // 128) * 8

    def _wzero_body(i, _):
        win_v[pl.ds(i * 16, 16)] = zero16
        return _
    lax.fori_loop(0, nz, _wzero_body, None)

    def _accum_body(i, _):
        sl = pl.ds(i * 16, 16)
        iv = idx1_v[sl] - lo18
        plsc.addupdate_scatter(win_v, [iv], val1_v[sl])
        return _
    lax.fori_loop(0, _E_PER_TILE_P1 // 16, _accum_body, None)

    iota16 = lax.iota(jnp.int32, 16)
    nck = (hi1 - lo18 + 128) // 128

    def _mk_row(t, _):
        for u in range(8):
            milist_v[t, pl.ds(u * 16, 16)] = iota16 + (lo18 + t * 128 + u * 16)
        return _
    lax.fori_loop(0, nck, _mk_row, None)

    def _mg_grp(g, _):
        jhi = jnp.minimum((g + 1) * _GRP, nck)

        def _fire(j, _):
            pltpu.async_copy(win_v.at[pl.ds(j * 128, 128)],
                             acc_sh.at[milist_v.at[j]], sem_sc, add=True)
            return _
        lax.fori_loop(g * _GRP, jhi, _fire, None)

        def _drain(j, _):
            pltpu.make_async_copy(win_v.at[pl.ds(j * 128, 128)],
                                  acc_sh.at[milist_v.at[j]], sem_sc).wait()
            return _
        lax.fori_loop(g * _GRP, jhi, _drain, None)
        return _
    lax.fori_loop(0, (nck + _GRP - 1) // _GRP, _mg_grp, None)
    plsc.subcore_barrier()

    pltpu.sync_copy(acc_sh.at[pl.ds(s * _SEG_PER_TILE, _SEG_PER_TILE)], seg_v)

    def _inv_body(i, _):
        seg_v[pl.ds(i * 16, 16)] = 1.0 / seg_v[pl.ds(i * 16, 16)]
        return _
    lax.fori_loop(0, _SEG_PER_TILE // 16, _inv_body, None)
    pltpu.sync_copy(seg_v, acc_sh.at[pl.ds(s * _SEG_PER_TILE, _SEG_PER_TILE)])
    plsc.subcore_barrier()

    pltpu.make_async_copy(idx_hbm.at[pl.ds(base2, _E_PER_TILE_P2)], idx_v,
                          sem_p2).wait()
    pltpu.make_async_copy(e_hbm.at[pl.ds(base2, _E_PER_TILE_P2)], val_v,
                          sem_p2).wait()
    lo = jnp.min(idx_v[pl.ds(0, 16)].astype(jnp.float32)).astype(jnp.int32)
    hi = jnp.max(idx_v[pl.ds(_E_PER_TILE_P2 - 16, 16)]
                 .astype(jnp.float32)).astype(jnp.int32)
    lo8 = jnp.bitwise_and(lo, -8)
    nch = (hi - lo8 + _WCH) // _WCH

    def _win_body(t, _):
        off = pl.multiple_of(lo8 + t * _WCH, 8)
        pltpu.sync_copy(acc_sh.at[pl.ds(off, _WCH)],
                        rng_v.at[pl.ds(t * _WCH, _WCH)])
        return _
    lax.fori_loop(0, nch, _win_body, None)

    def _mul_body(i, _):
        sl = pl.ds(i * 16, 16)
        iv = idx_v[sl] - lo8
        val_v[sl] = val_v[sl] * plsc.load_gather(rng_v, [iv])
        return _
    lax.fori_loop(0, _E_PER_TILE_P2 // 16, _mul_body, None)
    pltpu.sync_copy(val_v, out_hbm.at[pl.ds(base2, _E_PER_TILE_P2)])


def _sc_softmax(e, index):
    mesh = plsc.VectorSubcoreMesh(core_axis_name="c", subcore_axis_name="s")
    fn = functools.partial(
        pl.kernel,
        mesh=mesh,
        compiler_params=pltpu.CompilerParams(needs_layout_passes=False),
        out_type=jax.ShapeDtypeStruct((N,), jnp.float32),
        scratch_types=[
            pltpu.VMEM((_E_PER_TILE_P1,), jnp.int32),
            pltpu.VMEM((_E_PER_TILE_P1,), jnp.float32),
            pltpu.VMEM((_WCAP,), jnp.float32),
            pltpu.VMEM((_WCAP // 128, 128), jnp.int32),
            pltpu.VMEM((_E_PER_TILE_P2,), jnp.int32),
            pltpu.VMEM((_WCAP,), jnp.float32),
            pltpu.VMEM((_E_PER_TILE_P2,), jnp.float32),
            pltpu.VMEM((_SEG_PER_TILE,), jnp.float32),
            pltpu.VMEM_SHARED((SPAD,), jnp.float32),
            pltpu.SemaphoreType.DMA,
            pltpu.SemaphoreType.DMA,
            pltpu.SemaphoreType.DMA,
        ],
    )(_sc_softmax_body)
    return fn(e, index)


def kernel(q, k, index):
    e = _tc_scores(q, k)
    return _sc_softmax(e, index)


# speedup vs baseline: 1.0455x; 1.0455x over previous
import functools

import jax
import jax.numpy as jnp
from jax import lax
from jax.experimental import pallas as pl
from jax.experimental.pallas import tpu as pltpu
from jax.experimental.pallas import tpu_sc as plsc

N = 320000
D = 128
NSEG = 10000
SPAD = 11264
TEMP_INV = 1.0 / 11.313708498984761

_BLK = 125
_NBLK = N // (D * _BLK)


def _tc_scores_body(q_ref, k_ref, o_ref):
    s = jnp.sum(q_ref[...] * k_ref[...], axis=3)
    o_ref[...] = jnp.exp(s * TEMP_INV)


def _tc_scores(q, k):
    q4 = q.reshape(_NBLK, _BLK, D, D)
    k4 = k.reshape(_NBLK, _BLK, D, D)
    out = pl.pallas_call(
        _tc_scores_body,
        grid=(_NBLK,),
        in_specs=[
            pl.BlockSpec((1, _BLK, D, D), lambda i: (i, 0, 0, 0)),
            pl.BlockSpec((1, _BLK, D, D), lambda i: (i, 0, 0, 0)),
        ],
        out_specs=pl.BlockSpec((1, _BLK, D), lambda i: (i, 0, 0)),
        out_shape=jax.ShapeDtypeStruct((_NBLK, _BLK, D), jnp.float32),
    )(q4, k4)
    return out.reshape(N)


_NTILES = 16
_CH = 80
_E_PER_TILE_P1 = N // _NTILES
_E_PER_TILE_P2 = N // (2 * _NTILES)
_P1_CHUNKS = _E_PER_TILE_P1 // _CH
_SEG_PER_TILE = SPAD // _NTILES
_GRP = 25
_WCH = 640
_WCAP = 10240


def _sc_softmax_body(e_hbm, idx_hbm, out_hbm,
                     idx2_v, val2_v, idx_v, rng_v, val_v, seg_v, acc_sh,
                     sem_ld, sem_sc, sem_p2):
    c = lax.axis_index("c")
    s = lax.axis_index("s")

    zero16 = jnp.zeros((16,), jnp.float32)

    wid = c * _NTILES + s
    base2 = wid * _E_PER_TILE_P2
    pltpu.async_copy(idx_hbm.at[pl.ds(base2, _E_PER_TILE_P2)], idx_v, sem_p2)
    pltpu.async_copy(e_hbm.at[pl.ds(base2, _E_PER_TILE_P2)], val_v, sem_p2)

    def _zero_body(i, _):
        seg_v[pl.ds(i * 16, 16)] = zero16
        return _
    lax.fori_loop(0, _SEG_PER_TILE // 16, _zero_body, None)
    pltpu.sync_copy(seg_v, acc_sh.at[pl.ds(s * _SEG_PER_TILE, _SEG_PER_TILE)])

    base1 = s * _E_PER_TILE_P1

    def _ld_grp(g, _):
        def _fire(j, _):
            off = base1 + j * _CH
            pltpu.async_copy(idx_hbm.at[pl.ds(off, _CH)], idx2_v.at[j], sem_ld)
            pltpu.async_copy(e_hbm.at[pl.ds(off, _CH)], val2_v.at[j], sem_ld)
            return _
        lax.fori_loop(g * _GRP, (g + 1) * _GRP, _fire, None)

        def _drain(j, _):
            off = base1 + j * _CH
            pltpu.make_async_copy(idx_hbm.at[pl.ds(off, _CH)], idx2_v.at[j],
                                  sem_ld).wait()
            pltpu.make_async_copy(e_hbm.at[pl.ds(off, _CH)], val2_v.at[j],
                                  sem_ld).wait()
            return _
        lax.fori_loop(g * _GRP, (g + 1) * _GRP, _drain, None)
        return _
    lax.fori_loop(0, _P1_CHUNKS // _GRP, _ld_grp, None)
    plsc.subcore_barrier()

    def _sc_grp(g, _):
        def _fire(j, _):
            pltpu.async_copy(val2_v.at[j], acc_sh.at[idx2_v.at[j]], sem_sc,
                             add=True)
            return _
        lax.fori_loop(g * _GRP, (g + 1) * _GRP, _fire, None)

        def _drain(j, _):
            pltpu.make_async_copy(val2_v.at[j], acc_sh.at[idx2_v.at[j]],
                                  sem_sc).wait()
            return _
        lax.fori_loop(g * _GRP, (g + 1) * _GRP, _drain, None)
        return _
    lax.fori_loop(0, _P1_CHUNKS // _GRP, _sc_grp, None)
    plsc.subcore_barrier()

    pltpu.sync_copy(acc_sh.at[pl.ds(s * _SEG_PER_TILE, _SEG_PER_TILE)], seg_v)

    def _inv_body(i, _):
        seg_v[pl.ds(i * 16, 16)] = 1.0 / seg_v[pl.ds(i * 16, 16)]
        return _
    lax.fori_loop(0, _SEG_PER_TILE // 16, _inv_body, None)
    pltpu.sync_copy(seg_v, acc_sh.at[pl.ds(s * _SEG_PER_TILE, _SEG_PER_TILE)])
    plsc.subcore_barrier()

    pltpu.make_async_copy(idx_hbm.at[pl.ds(base2, _E_PER_TILE_P2)], idx_v,
                          sem_p2).wait()
    pltpu.make_async_copy(e_hbm.at[pl.ds(base2, _E_PER_TILE_P2)], val_v,
                          sem_p2).wait()
    lo = jnp.min(idx_v[pl.ds(0, 16)].astype(jnp.float32)).astype(jnp.int32)
    hi = jnp.max(idx_v[pl.ds(_E_PER_TILE_P2 - 16, 16)]
                 .astype(jnp.float32)).astype(jnp.int32)
    lo8 = jnp.bitwise_and(lo, -8)
    nch = (hi - lo8 + _WCH) // _WCH

    def _win_body(t, _):
        off = pl.multiple_of(lo8 + t * _WCH, 8)
        pltpu.sync_copy(acc_sh.at[pl.ds(off, _WCH)],
                        rng_v.at[pl.ds(t * _WCH, _WCH)])
        return _
    lax.fori_loop(0, nch, _win_body, None)

    def _mul_body(i, _):
        sl = pl.ds(i * 16, 16)
        iv = idx_v[sl] - lo8
        val_v[sl] = val_v[sl] * plsc.load_gather(rng_v, [iv])
        return _
    lax.fori_loop(0, _E_PER_TILE_P2 // 16, _mul_body, None)
    pltpu.sync_copy(val_v, out_hbm.at[pl.ds(base2, _E_PER_TILE_P2)])


def _sc_softmax(e, index):
    mesh = plsc.VectorSubcoreMesh(core_axis_name="c", subcore_axis_name="s")
    fn = functools.partial(
        pl.kernel,
        mesh=mesh,
        compiler_params=pltpu.CompilerParams(needs_layout_passes=False),
        out_type=jax.ShapeDtypeStruct((N,), jnp.float32),
        scratch_types=[
            pltpu.VMEM((_P1_CHUNKS, _CH), jnp.int32),
            pltpu.VMEM((_P1_CHUNKS, _CH), jnp.float32),
            pltpu.VMEM((_E_PER_TILE_P2,), jnp.int32),
            pltpu.VMEM((_WCAP,), jnp.float32),
            pltpu.VMEM((_E_PER_TILE_P2,), jnp.float32),
            pltpu.VMEM((_SEG_PER_TILE,), jnp.float32),
            pltpu.VMEM_SHARED((SPAD,), jnp.float32),
            pltpu.SemaphoreType.DMA,
            pltpu.SemaphoreType.DMA,
            pltpu.SemaphoreType.DMA,
        ],
    )(_sc_softmax_body)
    return fn(e, index)


def kernel(q, k, index):
    e = _tc_scores(q, k)
    return _sc_softmax(e, index)
